# baseline (speedup 1.0000x reference)
import functools

import jax
import jax.numpy as jnp
from jax import lax
from jax.experimental import pallas as pl
from jax.experimental.pallas import tpu as pltpu
from jax.experimental.pallas import tpu_sc as plsc

_DIM = 128
_HALF = _DIM // 2
_BATCH = 16384
_NC = 2
_NS = 16
_NW = _NC * _NS
_ROWS_PER_W = _BATCH // _NW
_PASS = 256
_NPASS = _ROWS_PER_W // _PASS
_LANES = 16

_mesh = plsc.VectorSubcoreMesh(core_axis_name="c", subcore_axis_name="s")


@functools.partial(
    pl.kernel,
    out_type=jax.ShapeDtypeStruct((_BATCH, _DIM), jnp.float32),
    mesh=_mesh,
    scratch_types=[
        pltpu.VMEM((_ROWS_PER_W + _LANES,), jnp.int32),
        pltpu.VMEM((_PASS, _DIM), jnp.float32),
        pltpu.VMEM((_PASS, _HALF), jnp.float32),
        pltpu.VMEM((_PASS, _HALF), jnp.float32),
        pltpu.SemaphoreType.DMA,
    ],
)
def _sc_complex_diag(emb_hbm, idx_hbm, real_hbm, imag_hbm, out_hbm,
                     idx_v, emb_v, rb_v, ib_v, sem):
    wid = lax.axis_index("s") * _NC + lax.axis_index("c")
    base = wid * _ROWS_PER_W

    pltpu.sync_copy(idx_hbm.at[pl.ds(base, _ROWS_PER_W)],
                    idx_v.at[pl.ds(0, _ROWS_PER_W)])

    for p in range(_NPASS):
        row0 = base + p * _PASS
        emb_cp = pltpu.async_copy(emb_hbm.at[pl.ds(row0, _PASS)], emb_v, sem)

        @pl.loop(0, _PASS)
        def _(r):
            k = idx_v[pl.ds(p * _PASS + r, _LANES)][0]
            pltpu.async_copy(real_hbm.at[k], rb_v.at[r], sem)
            pltpu.async_copy(imag_hbm.at[k], ib_v.at[r], sem)

        dummy = real_hbm.at[pl.ds(0, _PASS)]
        pltpu.make_async_copy(dummy, rb_v, sem).wait()
        pltpu.make_async_copy(dummy, ib_v, sem).wait()
        emb_cp.wait()

        @pl.loop(0, _PASS)
        def _(r):
            for c in range(_HALF // _LANES):
                lo = pl.ds(c * _LANES, _LANES)
                hi = pl.ds(_HALF + c * _LANES, _LANES)
                ra = emb_v[r, lo]
                ia = emb_v[r, hi]
                rb = rb_v[r, lo]
                ib = ib_v[r, lo]
                emb_v[r, lo] = ra * rb - ia * ib
                emb_v[r, hi] = ra * ib + ia * rb

        pltpu.sync_copy(emb_v, out_hbm.at[pl.ds(row0, _PASS)])


def kernel(embeddings, operator_idxs, real, imag):
    idx = operator_idxs.astype(jnp.int32)
    return _sc_complex_diag(embeddings, idx, real, imag)
